# Initial kernel scaffold; baseline (speedup 1.0000x reference)
import jax
import jax.numpy as jnp
from jax.experimental import pallas as pl


def kernel(x, embedding_weight):
    raise NotImplementedError("write your pallas kernel here")


# speedup vs baseline: 1.0038x; 1.0038x over previous
import functools

import jax
import jax.numpy as jnp
from jax import lax
from jax.experimental import pallas as pl
from jax.experimental.pallas import tpu as pltpu
from jax.experimental.pallas import tpu_sc as plsc

N_ROWS = 16384
K_CODES = 8192
DIM = 32
ROWS = 256


def _vq_tc_kernel(x_ref, e_ref, idx_ref, losssum_ref, acc_ref):
    i = pl.program_id(0)
    x = x_ref[...]
    e = e_ref[...]

    a = jnp.sum(x * x, axis=1, keepdims=True)
    b = lax.dot_general(x, e, (((1,), (1,)), ((), ())),
                        preferred_element_type=jnp.float32)
    c = jnp.sum(e * e, axis=1)[None, :]
    dist = (a - 2.0 * b) + c

    minv = jnp.min(dist, axis=1, keepdims=True)
    cols = lax.broadcasted_iota(jnp.int32, dist.shape, 1)
    idx = jnp.min(jnp.where(dist == minv, cols, K_CODES), axis=1)
    idx_ref[...] = idx

    @pl.when(i == 0)
    def _init():
        acc_ref[0] = 0.0

    acc_ref[0] += jnp.sum(minv)

    @pl.when(i == pl.num_programs(0) - 1)
    def _fin():
        losssum_ref[0] = acc_ref[0]


def _tc_indices_and_losssum(flat_x, embedding_weight):
    grid = N_ROWS // ROWS
    return pl.pallas_call(
        _vq_tc_kernel,
        grid=(grid,),
        in_specs=[
            pl.BlockSpec((ROWS, DIM), lambda i: (i, 0)),
            pl.BlockSpec((K_CODES, DIM), lambda i: (0, 0)),
        ],
        out_specs=[
            pl.BlockSpec((ROWS,), lambda i: (i,)),
            pl.BlockSpec(memory_space=pltpu.SMEM),
        ],
        out_shape=[
            jax.ShapeDtypeStruct((N_ROWS,), jnp.int32),
            jax.ShapeDtypeStruct((1,), jnp.float32),
        ],
        scratch_shapes=[pltpu.SMEM((1,), jnp.float32)],
    )(flat_x, embedding_weight)


D_PAD = 128


def _sc_gather(table_padded, idx):
    info = plsc.get_sparse_core_info()
    nc, ns = info.num_cores, info.num_subcores
    nw = nc * ns
    b_per_w = N_ROWS // nw
    mesh = plsc.VectorSubcoreMesh(core_axis_name="c", subcore_axis_name="s")

    @functools.partial(
        pl.kernel, mesh=mesh,
        out_type=jax.ShapeDtypeStruct((N_ROWS, D_PAD), jnp.float32),
        scratch_types=[
            pltpu.VMEM((b_per_w,), jnp.int32),
            pltpu.VMEM((b_per_w, D_PAD), jnp.float32),
            pltpu.SemaphoreType.DMA,
        ],
    )
    def gather_k(table_hbm, idx_hbm, out_hbm, idx_v, rows_v, sem):
        wid = lax.axis_index("s") * nc + lax.axis_index("c")
        base = wid * b_per_w
        pltpu.sync_copy(idx_hbm.at[pl.ds(base, b_per_w)], idx_v)
        pltpu.async_copy(table_hbm.at[idx_v], rows_v, sem).wait()
        pltpu.sync_copy(rows_v, out_hbm.at[pl.ds(base, b_per_w)])

    return gather_k(table_padded, idx)


def kernel(x, embedding_weight):
    flat_x = x.reshape(-1, DIM)
    idx, loss_sum = _tc_indices_and_losssum(flat_x, embedding_weight)
    table_padded = jnp.pad(embedding_weight, ((0, 0), (0, D_PAD - DIM)))
    quantized = _sc_gather(table_padded, idx)[:, :DIM]
    m = loss_sum[0] / jnp.float32(N_ROWS * DIM)
    loss = m + 0.25 * m
    quantized_st = flat_x + (quantized - flat_x)
    return (quantized_st.reshape(x.shape), idx, loss.reshape(()))


# speedup vs baseline: 1.2219x; 1.2173x over previous
import functools

import jax
import jax.numpy as jnp
from jax import lax
from jax.experimental import pallas as pl
from jax.experimental.pallas import tpu as pltpu
from jax.experimental.pallas import tpu_sc as plsc

N_ROWS = 16384
K_CODES = 8192
DIM = 32
ROWS = 512


def _vq_tc_kernel(x_ref, e_ref, idx_ref, losssum_ref, acc_ref):
    i = pl.program_id(0)
    x = x_ref[...]
    e = e_ref[...]

    a = jnp.sum(x * x, axis=1, keepdims=True)
    b = lax.dot_general(x, -2.0 * e, (((1,), (1,)), ((), ())),
                        preferred_element_type=jnp.float32)
    c = jnp.sum(e * e, axis=1)[None, :]
    dist = (a + b) + c

    minv = jnp.min(dist, axis=1, keepdims=True)
    cols = lax.broadcasted_iota(jnp.int32, dist.shape, 1)
    idx = jnp.min(jnp.where(dist == minv, cols, K_CODES), axis=1)
    idx_ref[...] = idx

    @pl.when(i == 0)
    def _init():
        acc_ref[0] = 0.0

    acc_ref[0] += jnp.sum(minv)

    @pl.when(i == pl.num_programs(0) - 1)
    def _fin():
        losssum_ref[0] = acc_ref[0]


def _tc_indices_and_losssum(flat_x, embedding_weight):
    grid = N_ROWS // ROWS
    return pl.pallas_call(
        _vq_tc_kernel,
        grid=(grid,),
        in_specs=[
            pl.BlockSpec((ROWS, DIM), lambda i: (i, 0)),
            pl.BlockSpec((K_CODES, DIM), lambda i: (0, 0)),
        ],
        out_specs=[
            pl.BlockSpec((ROWS,), lambda i: (i,)),
            pl.BlockSpec(memory_space=pltpu.SMEM),
        ],
        out_shape=[
            jax.ShapeDtypeStruct((N_ROWS,), jnp.int32),
            jax.ShapeDtypeStruct((1,), jnp.float32),
        ],
        scratch_shapes=[pltpu.SMEM((1,), jnp.float32)],
    )(flat_x, embedding_weight)


D_PAD = 128


def _sc_gather(table_padded, idx):
    info = plsc.get_sparse_core_info()
    nc, ns = info.num_cores, info.num_subcores
    nw = nc * ns
    b_per_w = N_ROWS // nw
    mesh = plsc.VectorSubcoreMesh(core_axis_name="c", subcore_axis_name="s")

    @functools.partial(
        pl.kernel, mesh=mesh,
        out_type=jax.ShapeDtypeStruct((N_ROWS, D_PAD), jnp.float32),
        scratch_types=[
            pltpu.VMEM((b_per_w,), jnp.int32),
            pltpu.VMEM((b_per_w, D_PAD), jnp.float32),
            pltpu.SemaphoreType.DMA,
        ],
    )
    def gather_k(table_hbm, idx_hbm, out_hbm, idx_v, rows_v, sem):
        wid = lax.axis_index("s") * nc + lax.axis_index("c")
        base = wid * b_per_w
        pltpu.sync_copy(idx_hbm.at[pl.ds(base, b_per_w)], idx_v)
        pltpu.async_copy(table_hbm.at[idx_v], rows_v, sem).wait()
        pltpu.sync_copy(rows_v, out_hbm.at[pl.ds(base, b_per_w)])

    return gather_k(table_padded, idx)


def kernel(x, embedding_weight):
    flat_x = x.reshape(-1, DIM)
    idx, loss_sum = _tc_indices_and_losssum(flat_x, embedding_weight)
    table_padded = jnp.pad(embedding_weight, ((0, 0), (0, D_PAD - DIM)))
    quantized = _sc_gather(table_padded, idx)[:, :DIM]
    m = loss_sum[0] / jnp.float32(N_ROWS * DIM)
    loss = m + 0.25 * m
    quantized_st = flat_x + (quantized - flat_x)
    return (quantized_st.reshape(x.shape), idx, loss.reshape(()))


# speedup vs baseline: 1.2803x; 1.0478x over previous
import functools

import jax
import jax.numpy as jnp
from jax import lax
from jax.experimental import pallas as pl
from jax.experimental.pallas import tpu as pltpu
from jax.experimental.pallas import tpu_sc as plsc

N_ROWS = 16384
K_CODES = 8192
DIM = 32
ROWS = 1024


def _vq_tc_kernel(x_ref, e_ref, idx_ref, losssum_ref, acc_ref):
    i = pl.program_id(0)
    x = x_ref[...]
    e = e_ref[...]

    a = jnp.sum(x * x, axis=1, keepdims=True)
    b = lax.dot_general(x, -2.0 * e, (((1,), (1,)), ((), ())),
                        preferred_element_type=jnp.float32)
    c = jnp.sum(e * e, axis=1)[None, :]
    dist = (a + b) + c

    minv = jnp.min(dist, axis=1, keepdims=True)
    cols = lax.broadcasted_iota(jnp.int32, dist.shape, 1)
    idx = jnp.min(jnp.where(dist == minv, cols, K_CODES), axis=1)
    idx_ref[...] = idx

    @pl.when(i == 0)
    def _init():
        acc_ref[0] = 0.0

    acc_ref[0] += jnp.sum(minv)

    @pl.when(i == pl.num_programs(0) - 1)
    def _fin():
        losssum_ref[0] = acc_ref[0]


def _tc_indices_and_losssum(flat_x, embedding_weight):
    grid = N_ROWS // ROWS
    return pl.pallas_call(
        _vq_tc_kernel,
        grid=(grid,),
        in_specs=[
            pl.BlockSpec((ROWS, DIM), lambda i: (i, 0)),
            pl.BlockSpec((K_CODES, DIM), lambda i: (0, 0)),
        ],
        out_specs=[
            pl.BlockSpec((ROWS,), lambda i: (i,)),
            pl.BlockSpec(memory_space=pltpu.SMEM),
        ],
        out_shape=[
            jax.ShapeDtypeStruct((N_ROWS,), jnp.int32),
            jax.ShapeDtypeStruct((1,), jnp.float32),
        ],
        scratch_shapes=[pltpu.SMEM((1,), jnp.float32)],
    )(flat_x, embedding_weight)


D_PAD = 128


def _sc_gather(table_padded, idx):
    info = plsc.get_sparse_core_info()
    nc, ns = info.num_cores, info.num_subcores
    nw = nc * ns
    b_per_w = N_ROWS // nw
    mesh = plsc.VectorSubcoreMesh(core_axis_name="c", subcore_axis_name="s")

    @functools.partial(
        pl.kernel, mesh=mesh,
        out_type=jax.ShapeDtypeStruct((N_ROWS, D_PAD), jnp.float32),
        scratch_types=[
            pltpu.VMEM((b_per_w,), jnp.int32),
            pltpu.VMEM((b_per_w, D_PAD), jnp.float32),
            pltpu.SemaphoreType.DMA,
        ],
    )
    def gather_k(table_hbm, idx_hbm, out_hbm, idx_v, rows_v, sem):
        wid = lax.axis_index("s") * nc + lax.axis_index("c")
        base = wid * b_per_w
        pltpu.sync_copy(idx_hbm.at[pl.ds(base, b_per_w)], idx_v)
        pltpu.async_copy(table_hbm.at[idx_v], rows_v, sem).wait()
        pltpu.sync_copy(rows_v, out_hbm.at[pl.ds(base, b_per_w)])

    return gather_k(table_padded, idx)


def kernel(x, embedding_weight):
    flat_x = x.reshape(-1, DIM)
    idx, loss_sum = _tc_indices_and_losssum(flat_x, embedding_weight)
    table_padded = jnp.pad(embedding_weight, ((0, 0), (0, D_PAD - DIM)))
    quantized = _sc_gather(table_padded, idx)[:, :DIM]
    m = loss_sum[0] / jnp.float32(N_ROWS * DIM)
    loss = m + 0.25 * m
    quantized_st = flat_x + (quantized - flat_x)
    return (quantized_st.reshape(x.shape), idx, loss.reshape(()))
